# baseline (speedup 1.0000x reference)
import jax
import jax.numpy as jnp
from jax import lax
from jax.experimental import pallas as pl
from jax.experimental.pallas import tpu as pltpu
from jax.experimental.pallas import tpu_sc as plsc

B, N, HW, D = 32, 1000, 16384, 64
NC, NS = 2, 16
NW = NC * NS
NPAD = 1024
HB = 16
NCH = 16

GB = 4


def _make_gather_body(half):
    def _gather_body(table, ind, out, idx_v, buf_a, buf_b, stage,
                     sem_a, sem_b):
        wid = lax.axis_index("s") * NC + lax.axis_index("c")
        b = half * HB + wid // 2
        ob = wid // 2
        dbase = (wid % 2) * 4

        for nb in range(8):
            pltpu.sync_copy(ind.at[b, nb], idx_v.at[pl.ds(nb * 128, 128)])

        bufs = (buf_a, buf_b)
        sems = (sem_a, sem_b)

        def start(c):
            db, rr = dbase + c // 4, c % 4
            pltpu.make_async_copy(
                table.at[b, db, :, pl.ds(rr * 2, 2), :],
                bufs[c % 2], sems[c % 2]).start()

        start(0)
        for c in range(NCH):
            if c + 1 < NCH:
                start(c + 1)
            db, rr = dbase + c // 4, c % 4
            pltpu.make_async_copy(
                table.at[b, db, :, pl.ds(rr * 2, 2), :],
                bufs[c % 2], sems[c % 2]).wait()
            rb = bufs[c % 2]

            def grp(g, _, rb=rb):
                h = idx_v[pl.ds(g * 16, 16)]
                hb = jnp.right_shift(h, 7)
                hl = jnp.bitwise_and(h, 127)
                nb = g // 8
                lo = (g % 8) * 16
                for r in range(2):
                    val = plsc.load_gather(
                        rb, [hb, jnp.full((16,), r, jnp.int32), hl])
                    stage[nb, r, pl.ds(lo, 16)] = val
                return 0

            lax.fori_loop(0, 64, grp, 0)
            pltpu.sync_copy(stage, out.at[ob, db, :, pl.ds(rr * 2, 2), :])
    return _gather_body


def _make_loss_body(half):
    def _loss_body(pred_ref, targ_ref, mask_ref, out_ref):
        @pl.when(pl.program_id(0) == 0)
        def _init():
            out_ref[...] = jnp.zeros_like(out_ref)

        p = pred_ref[:, :, :N]
        t = targ_ref[...]
        m = jnp.broadcast_to(mask_ref[...], (GB, D, N))
        pos = t == 1.0
        arg = jnp.where(pos, p, 1.0 - p)
        w = jnp.where(pos, jnp.float32(1.5), jnp.float32(1.0))
        v = w * jnp.log(arg) * m
        lane0 = lax.broadcasted_iota(jnp.int32, (1, 128), 1) == 0
        out_ref[0:1, :] += jnp.where(lane0, jnp.sum(v), 0.0)
        out_ref[1:2, :] += jnp.where(lane0, jnp.sum(m), 0.0)
    return _loss_body


@jax.jit
def _mask_loss(table5, ind5, mask3, targ_t):
    mesh = plsc.VectorSubcoreMesh(core_axis_name="c", subcore_axis_name="s")
    total = jnp.zeros((2, 128), jnp.float32)
    for half in range(2):
        pred5 = pl.kernel(
            _make_gather_body(half),
            out_type=jax.ShapeDtypeStruct((HB, 8, 8, 8, 128), jnp.float32),
            mesh=mesh,
            compiler_params=pltpu.CompilerParams(
                needs_layout_passes=False, use_tc_tiling_on_sc=True),
            scratch_types=[
                pltpu.VMEM((NPAD,), jnp.int32),
                pltpu.VMEM((128, 2, 128), jnp.float32),
                pltpu.VMEM((128, 2, 128), jnp.float32),
                pltpu.VMEM((8, 2, 128), jnp.float32),
                pltpu.SemaphoreType.DMA,
                pltpu.SemaphoreType.DMA,
            ],
        )(table5, ind5)

        pred3 = pred5.transpose(0, 1, 3, 2, 4).reshape(HB, D, NPAD)

        parts = pl.pallas_call(
            _make_loss_body(half),
            grid=(HB // GB,),
            in_specs=[
                pl.BlockSpec((GB, D, NPAD), lambda i: (i, 0, 0)),
                pl.BlockSpec((GB, D, N),
                             lambda i, half=half: (i + half * (HB // GB),
                                                   0, 0)),
                pl.BlockSpec((GB, 1, N),
                             lambda i, half=half: (i + half * (HB // GB),
                                                   0, 0)),
            ],
            out_specs=pl.BlockSpec((2, 128), lambda i: (0, 0)),
            out_shape=jax.ShapeDtypeStruct((2, 128), jnp.float32),
        )(pred3, targ_t, mask3)
        total = total + parts

    loss = 0.0 - jnp.sum(total[0, :])
    num = jnp.sum(total[1, :])
    return jnp.where(num > 0, loss / num, loss)


def kernel(output, mask, ind, target):
    table5 = output.reshape(B, 128, 128, 8, 8).transpose(0, 3, 1, 4, 2)
    ind5 = jnp.pad(ind.astype(jnp.int32), ((0, 0), (0, NPAD - N))
                   ).reshape(B, 8, 128)
    targ_t = target.transpose(0, 2, 3, 1).reshape(B, D, N)
    mask3 = mask.astype(jnp.float32).reshape(B, 1, N)
    return _mask_loss(table5, ind5, mask3, targ_t)


# speedup vs baseline: 1.0767x; 1.0767x over previous
import jax
import jax.numpy as jnp
from jax import lax
from jax.experimental import pallas as pl
from jax.experimental.pallas import tpu as pltpu
from jax.experimental.pallas import tpu_sc as plsc

B, N, HW, D = 32, 1000, 16384, 64
NC, NS = 2, 16
NW = NC * NS
NPAD = 1024
NCH = 32

GB = 4


def _gather_body(table, ind, out, idx_v, buf_a, buf_b, stage, sem_a, sem_b):
    wid = lax.axis_index("s") * NC + lax.axis_index("c")

    for nb in range(8):
        pltpu.sync_copy(ind.at[wid, nb], idx_v.at[pl.ds(nb * 128, 128)])

    bufs = (buf_a, buf_b)
    sems = (sem_a, sem_b)

    def start(c):
        db, rr = c // 4, c % 4
        pltpu.make_async_copy(
            table.at[wid, db, :, pl.ds(rr * 2, 2), :],
            bufs[c % 2], sems[c % 2]).start()

    start(0)
    for c in range(NCH):
        if c + 1 < NCH:
            start(c + 1)
        db, rr = c // 4, c % 4
        pltpu.make_async_copy(
            table.at[wid, db, :, pl.ds(rr * 2, 2), :],
            bufs[c % 2], sems[c % 2]).wait()
        rb = bufs[c % 2]

        def grp(g, _, rb=rb):
            h = idx_v[pl.ds(g * 16, 16)]
            hb = jnp.right_shift(h, 7)
            hl = jnp.bitwise_and(h, 127)
            nb = g // 8
            lo = (g % 8) * 16
            for r in range(2):
                val = plsc.load_gather(
                    rb, [hb, jnp.full((16,), r, jnp.int32), hl])
                stage[nb, r, pl.ds(lo, 16)] = val
            return 0

        lax.fori_loop(0, 64, grp, 0)
        pltpu.sync_copy(stage, out.at[wid, db, :, pl.ds(rr * 2, 2), :])


def _loss_body(pred_ref, targ_ref, mask_ref, out_ref):
    @pl.when(pl.program_id(0) == 0)
    def _init():
        out_ref[...] = jnp.zeros_like(out_ref)

    p = pred_ref[:, :, :N]
    t = targ_ref[...]
    m = jnp.broadcast_to(mask_ref[...], (GB, D, N))
    pos = t == 1.0
    arg = jnp.where(pos, p, 1.0 - p)
    w = jnp.where(pos, jnp.float32(1.5), jnp.float32(1.0))
    v = w * jnp.log(arg) * m
    lane0 = lax.broadcasted_iota(jnp.int32, (1, 128), 1) == 0
    out_ref[0:1, :] += jnp.where(lane0, jnp.sum(v), 0.0)
    out_ref[1:2, :] += jnp.where(lane0, jnp.sum(m), 0.0)


@jax.jit
def _mask_loss(table5, ind5, mask3, targ_t):
    mesh = plsc.VectorSubcoreMesh(core_axis_name="c", subcore_axis_name="s")
    pred5 = pl.kernel(
        _gather_body,
        out_type=jax.ShapeDtypeStruct((B, 8, 8, 8, 128), jnp.float32),
        mesh=mesh,
        compiler_params=pltpu.CompilerParams(
            needs_layout_passes=False, use_tc_tiling_on_sc=True),
        scratch_types=[
            pltpu.VMEM((NPAD,), jnp.int32),
            pltpu.VMEM((128, 2, 128), jnp.float32),
            pltpu.VMEM((128, 2, 128), jnp.float32),
            pltpu.VMEM((8, 2, 128), jnp.float32),
            pltpu.SemaphoreType.DMA,
            pltpu.SemaphoreType.DMA,
        ],
    )(table5, ind5)

    pred3 = pred5.transpose(0, 1, 3, 2, 4).reshape(B, D, NPAD)

    parts = pl.pallas_call(
        _loss_body,
        grid=(B // GB,),
        in_specs=[
            pl.BlockSpec((GB, D, NPAD), lambda i: (i, 0, 0)),
            pl.BlockSpec((GB, D, N), lambda i: (i, 0, 0)),
            pl.BlockSpec((GB, 1, N), lambda i: (i, 0, 0)),
        ],
        out_specs=pl.BlockSpec((2, 128), lambda i: (0, 0)),
        out_shape=jax.ShapeDtypeStruct((2, 128), jnp.float32),
    )(pred3, targ_t, mask3)

    loss = 0.0 - jnp.sum(parts[0, :])
    num = jnp.sum(parts[1, :])
    return jnp.where(num > 0, loss / num, loss)


def kernel(output, mask, ind, target):
    table5 = output.reshape(B, 128, 128, 8, 8).transpose(0, 3, 1, 4, 2)
    ind5 = jnp.pad(ind.astype(jnp.int32), ((0, 0), (0, NPAD - N))
                   ).reshape(B, 8, 128)
    targ_t = target.transpose(0, 2, 3, 1).reshape(B, D, N)
    mask3 = mask.astype(jnp.float32).reshape(B, 1, N)
    return _mask_loss(table5, ind5, mask3, targ_t)


# speedup vs baseline: 1.1020x; 1.0235x over previous
import jax
import jax.numpy as jnp
from jax import lax
from jax.experimental import pallas as pl
from jax.experimental.pallas import tpu as pltpu
from jax.experimental.pallas import tpu_sc as plsc

B, N, HW, D = 32, 1000, 16384, 64
NC, NS = 2, 16
NW = NC * NS
NPAD = 1024
NCH = 32

GB = 8


def _gather_body(table, ind, out, idx_v, buf_a, buf_b, stage, sem_a, sem_b):
    wid = lax.axis_index("s") * NC + lax.axis_index("c")

    for nb in range(8):
        pltpu.sync_copy(ind.at[wid, nb], idx_v.at[pl.ds(nb * 128, 128)])

    bufs = (buf_a, buf_b)
    sems = (sem_a, sem_b)

    def start(c):
        db, rr = c // 4, c % 4
        pltpu.make_async_copy(
            table.at[wid, db, :, pl.ds(rr * 2, 2), :],
            bufs[c % 2], sems[c % 2]).start()

    start(0)
    for c in range(NCH):
        if c + 1 < NCH:
            start(c + 1)
        db, rr = c // 4, c % 4
        pltpu.make_async_copy(
            table.at[wid, db, :, pl.ds(rr * 2, 2), :],
            bufs[c % 2], sems[c % 2]).wait()
        rb = bufs[c % 2]

        def grp(g, _, rb=rb):
            h = idx_v[pl.ds(g * 16, 16)]
            hb = jnp.right_shift(h, 7)
            hl = jnp.bitwise_and(h, 127)
            nb = g // 8
            lo = (g % 8) * 16
            for r in range(2):
                val = plsc.load_gather(
                    rb, [hb, jnp.full((16,), r, jnp.int32), hl])
                stage[nb, r, pl.ds(lo, 16)] = val
            return 0

        lax.fori_loop(0, 64, grp, 0)
        pltpu.sync_copy(stage, out.at[wid, db, :, pl.ds(rr * 2, 2), :])


def _loss_body(pred_ref, targ_ref, mask_ref, out_ref):
    @pl.when(pl.program_id(0) == 0)
    def _init():
        out_ref[...] = jnp.zeros_like(out_ref)

    p = pred_ref[:, :, :N]
    t = targ_ref[...]
    m = jnp.broadcast_to(mask_ref[...], (GB, D, N))
    pos = t == 1.0
    arg = jnp.where(pos, p, 1.0 - p)
    w = jnp.where(pos, jnp.float32(1.5), jnp.float32(1.0))
    v = w * jnp.log(arg) * m
    lane0 = lax.broadcasted_iota(jnp.int32, (1, 128), 1) == 0
    out_ref[0:1, :] += jnp.where(lane0, jnp.sum(v), 0.0)
    out_ref[1:2, :] += jnp.where(lane0, jnp.sum(m), 0.0)


@jax.jit
def _mask_loss(table5, ind5, mask3, targ_t):
    mesh = plsc.VectorSubcoreMesh(core_axis_name="c", subcore_axis_name="s")
    pred5 = pl.kernel(
        _gather_body,
        out_type=jax.ShapeDtypeStruct((B, 8, 8, 8, 128), jnp.float32),
        mesh=mesh,
        compiler_params=pltpu.CompilerParams(
            needs_layout_passes=False, use_tc_tiling_on_sc=True),
        scratch_types=[
            pltpu.VMEM((NPAD,), jnp.int32),
            pltpu.VMEM((128, 2, 128), jnp.float32),
            pltpu.VMEM((128, 2, 128), jnp.float32),
            pltpu.VMEM((8, 2, 128), jnp.float32),
            pltpu.SemaphoreType.DMA,
            pltpu.SemaphoreType.DMA,
        ],
    )(table5, ind5)

    pred3 = pred5.transpose(0, 1, 3, 2, 4).reshape(B, D, NPAD)

    parts = pl.pallas_call(
        _loss_body,
        grid=(B // GB,),
        in_specs=[
            pl.BlockSpec((GB, D, NPAD), lambda i: (i, 0, 0)),
            pl.BlockSpec((GB, D, N), lambda i: (i, 0, 0)),
            pl.BlockSpec((GB, 1, N), lambda i: (i, 0, 0)),
        ],
        out_specs=pl.BlockSpec((2, 128), lambda i: (0, 0)),
        out_shape=jax.ShapeDtypeStruct((2, 128), jnp.float32),
    )(pred3, targ_t, mask3)

    loss = 0.0 - jnp.sum(parts[0, :])
    num = jnp.sum(parts[1, :])
    return jnp.where(num > 0, loss / num, loss)


def kernel(output, mask, ind, target):
    table5 = output.reshape(B, 128, 128, 8, 8).transpose(0, 3, 1, 4, 2)
    ind5 = jnp.pad(ind.astype(jnp.int32), ((0, 0), (0, NPAD - N))
                   ).reshape(B, 8, 128)
    targ_t = target.transpose(0, 2, 3, 1).reshape(B, D, N)
    mask3 = mask.astype(jnp.float32).reshape(B, 1, N)
    return _mask_loss(table5, ind5, mask3, targ_t)


# speedup vs baseline: 1.1055x; 1.0032x over previous
import jax
import jax.numpy as jnp
from jax import lax
from jax.experimental import pallas as pl
from jax.experimental.pallas import tpu as pltpu
from jax.experimental.pallas import tpu_sc as plsc

B, N, HW, D = 32, 1000, 16384, 64
NC, NS = 2, 16
NW = NC * NS
NPAD = 1024
NCH = 32

GB = 16


def _gather_body(table, ind, out, idx_v, buf_a, buf_b, stage, sem_a, sem_b):
    wid = lax.axis_index("s") * NC + lax.axis_index("c")

    for nb in range(8):
        pltpu.sync_copy(ind.at[wid, nb], idx_v.at[pl.ds(nb * 128, 128)])

    bufs = (buf_a, buf_b)
    sems = (sem_a, sem_b)

    def start(c):
        db, rr = c // 4, c % 4
        pltpu.make_async_copy(
            table.at[wid, db, :, pl.ds(rr * 2, 2), :],
            bufs[c % 2], sems[c % 2]).start()

    start(0)
    for c in range(NCH):
        if c + 1 < NCH:
            start(c + 1)
        db, rr = c // 4, c % 4
        pltpu.make_async_copy(
            table.at[wid, db, :, pl.ds(rr * 2, 2), :],
            bufs[c % 2], sems[c % 2]).wait()
        rb = bufs[c % 2]

        def grp(g, _, rb=rb):
            h = idx_v[pl.ds(g * 16, 16)]
            hb = jnp.right_shift(h, 7)
            hl = jnp.bitwise_and(h, 127)
            nb = g // 8
            lo = (g % 8) * 16
            for r in range(2):
                val = plsc.load_gather(
                    rb, [hb, jnp.full((16,), r, jnp.int32), hl])
                stage[nb, r, pl.ds(lo, 16)] = val
            return 0

        lax.fori_loop(0, 64, grp, 0)
        pltpu.sync_copy(stage, out.at[wid, db, :, pl.ds(rr * 2, 2), :])


def _loss_body(pred_ref, targ_ref, mask_ref, out_ref):
    @pl.when(pl.program_id(0) == 0)
    def _init():
        out_ref[...] = jnp.zeros_like(out_ref)

    p = pred_ref[:, :, :N]
    t = targ_ref[...]
    m = jnp.broadcast_to(mask_ref[...], (GB, D, N))
    pos = t == 1.0
    arg = jnp.where(pos, p, 1.0 - p)
    w = jnp.where(pos, jnp.float32(1.5), jnp.float32(1.0))
    v = w * jnp.log(arg) * m
    lane0 = lax.broadcasted_iota(jnp.int32, (1, 128), 1) == 0
    out_ref[0:1, :] += jnp.where(lane0, jnp.sum(v), 0.0)
    out_ref[1:2, :] += jnp.where(lane0, jnp.sum(m), 0.0)


@jax.jit
def _mask_loss(table5, ind5, mask3, targ_t):
    mesh = plsc.VectorSubcoreMesh(core_axis_name="c", subcore_axis_name="s")
    pred5 = pl.kernel(
        _gather_body,
        out_type=jax.ShapeDtypeStruct((B, 8, 8, 8, 128), jnp.float32),
        mesh=mesh,
        compiler_params=pltpu.CompilerParams(
            needs_layout_passes=False, use_tc_tiling_on_sc=True),
        scratch_types=[
            pltpu.VMEM((NPAD,), jnp.int32),
            pltpu.VMEM((128, 2, 128), jnp.float32),
            pltpu.VMEM((128, 2, 128), jnp.float32),
            pltpu.VMEM((8, 2, 128), jnp.float32),
            pltpu.SemaphoreType.DMA,
            pltpu.SemaphoreType.DMA,
        ],
    )(table5, ind5)

    pred3 = pred5.transpose(0, 1, 3, 2, 4).reshape(B, D, NPAD)

    parts = pl.pallas_call(
        _loss_body,
        grid=(B // GB,),
        in_specs=[
            pl.BlockSpec((GB, D, NPAD), lambda i: (i, 0, 0)),
            pl.BlockSpec((GB, D, N), lambda i: (i, 0, 0)),
            pl.BlockSpec((GB, 1, N), lambda i: (i, 0, 0)),
        ],
        out_specs=pl.BlockSpec((2, 128), lambda i: (0, 0)),
        out_shape=jax.ShapeDtypeStruct((2, 128), jnp.float32),
    )(pred3, targ_t, mask3)

    loss = 0.0 - jnp.sum(parts[0, :])
    num = jnp.sum(parts[1, :])
    return jnp.where(num > 0, loss / num, loss)


def kernel(output, mask, ind, target):
    table5 = output.reshape(B, 128, 128, 8, 8).transpose(0, 3, 1, 4, 2)
    ind5 = jnp.pad(ind.astype(jnp.int32), ((0, 0), (0, NPAD - N))
                   ).reshape(B, 8, 128)
    targ_t = target.transpose(0, 2, 3, 1).reshape(B, D, N)
    mask3 = mask.astype(jnp.float32).reshape(B, 1, N)
    return _mask_loss(table5, ind5, mask3, targ_t)
